# baseline (speedup 1.0000x reference)
import jax
import jax.numpy as jnp
from jax import lax
from jax.experimental import pallas as pl
from jax.experimental.pallas import tpu as pltpu
from jax.experimental.pallas import tpu_sc as plsc

B = 8
C = 192
HW = 224 * 224
NC = 2
NS = 16
L = 16
NW = NC * NS
W = 128
KPB = HW // W
NV = W // L
NITEM = B * KPB // NW
CT = C // 8
AT = B * C // 64


def _kernel_body(x_hbm, t_hbm, out_hbm, buf0, buf1, acc, tv, sem0, sem1):
    cid = lax.axis_index("c")
    sid = lax.axis_index("s")
    wid = sid * NC + cid

    pltpu.sync_copy(t_hbm, tv)
    tgt = tv[...]

    def copy_for(i, buf, sem):
        t = wid * NITEM + i
        b = t // KPB
        k = t % KPB
        src = x_hbm.at[pl.ds(b * CT, CT), :, pl.ds(k * W, W)]
        return pltpu.make_async_copy(src, buf, sem)

    zero = jnp.zeros((L,), jnp.float32)

    def zbody(t, _):
        for s in range(8):
            for g in range(8):
                acc[t, s, pl.ds(g * L, L)] = zero
        return 0

    lax.fori_loop(0, AT, zbody, 0)

    def compute(buf, i):
        b = (wid * NITEM + i) // KPB

        maxv = [buf[0, 0, pl.ds(v * L, L)] for v in range(NV)]
        idxv = [jnp.zeros((L,), jnp.int32) for _ in range(NV)]

        def cbody(tc, carry):
            mv = list(carry[:NV])
            iv = list(carry[NV:])
            for cc in range(8):
                cvec = jnp.full((L,), tc * 8 + cc, jnp.int32)
                for v in range(NV):
                    val = buf[tc, cc, pl.ds(v * L, L)]
                    gt = val > mv[v]
                    iv[v] = jnp.where(gt, cvec, iv[v])
                    mv[v] = jnp.maximum(val, mv[v])
            return tuple(mv) + tuple(iv)

        carry = lax.fori_loop(0, CT, cbody, tuple(maxv) + tuple(idxv))
        ones = jnp.full((L,), 1.0, jnp.float32)
        zeros = jnp.zeros((L,), jnp.float32)
        mf = [
            jnp.where(carry[NV + v] == tgt, ones, zeros) for v in range(NV)
        ]

        def abody(tc, _):
            at = b * 3 + tc // 8
            asub = tc % 8
            for cc in range(8):
                s = buf[tc, cc, pl.ds(0, L)] * mf[0]
                for v in range(1, NV):
                    s = s + buf[tc, cc, pl.ds(v * L, L)] * mf[v]
                plsc.addupdate(acc.at[at, asub, pl.ds(cc * L, L)], s)
            return 0

        lax.fori_loop(0, CT, abody, 0)

    copy_for(0, buf0, sem0).start()

    def loop_body(j, _):
        t0 = 2 * j
        t1 = t0 + 1
        t2 = t0 + 2
        copy_for(t0, buf0, sem0).wait()
        copy_for(t1, buf1, sem1).start()
        compute(buf0, t0)
        copy_for(t1, buf1, sem1).wait()

        @pl.when(t2 < NITEM)
        def _():
            copy_for(t2, buf0, sem0).start()

        compute(buf1, t1)
        return 0

    lax.fori_loop(0, NITEM // 2, loop_body, 0)

    pltpu.sync_copy(acc, out_hbm.at[wid])


@jax.jit
def _class_filter_sc(x3, tvec16):
    mesh = plsc.VectorSubcoreMesh(core_axis_name="c", subcore_axis_name="s")
    partials = pl.kernel(
        _kernel_body,
        out_type=jax.ShapeDtypeStruct((NW, AT, 8, 128), jnp.float32),
        mesh=mesh,
        scratch_types=[
            pltpu.VMEM((CT, 8, W), jnp.float32),
            pltpu.VMEM((CT, 8, W), jnp.float32),
            pltpu.VMEM((AT, 8, 128), jnp.float32),
            pltpu.VMEM((L,), jnp.int32),
            pltpu.SemaphoreType.DMA,
            pltpu.SemaphoreType.DMA,
        ],
    )(x3, tvec16)
    return jnp.sum(partials.reshape(NW, B, C, L), axis=(0, 3))


def kernel(logits_batch, target):
    x3 = logits_batch.reshape(B * C // 8, 8, HW)
    tvec16 = jnp.full((L,), target, jnp.int32)
    return _class_filter_sc(x3, tvec16)


# speedup vs baseline: 1.7520x; 1.7520x over previous
import jax
import jax.numpy as jnp
from jax import lax
from jax.experimental import pallas as pl
from jax.experimental.pallas import tpu as pltpu
from jax.experimental.pallas import tpu_sc as plsc

B = 8
C = 192
H = 224
WID = 224
NC = 2
NS = 16
L = 16
NW = NC * NS
RT = H // 8
NITEM = B * RT // NW
KC = 16
NKC = C // KC
NG = WID // L
NSEQ = NITEM * 2 * NKC
AT = B * C // 64


def _kernel_body(x_hbm, t_hbm, out_hbm, buf0, buf1, acc, mxb, idb, mfb, tv,
                 sem0, sem1):
    cid = lax.axis_index("c")
    sid = lax.axis_index("s")
    wid = sid * NC + cid

    pltpu.sync_copy(t_hbm, tv)
    tgt = tv[...]

    def copy_for(s, buf, sem):
        item = wid * NITEM + s // (2 * NKC)
        kc = s % NKC
        b = item // RT
        rt = item % RT
        src = x_hbm.at[pl.ds(b * C + kc * KC, KC), pl.ds(rt * 8, 8), :]
        return pltpu.make_async_copy(src, buf, sem)

    zero = jnp.zeros((L,), jnp.float32)

    def zbody(t, _):
        for s in range(8):
            for g in range(8):
                acc[t, s, pl.ds(g * L, L)] = zero
        return 0

    lax.fori_loop(0, AT, zbody, 0)

    neginf = jnp.full((L,), -jnp.inf, jnp.float32)
    zeroi = jnp.zeros((L,), jnp.int32)
    ones = jnp.full((L,), 1.0, jnp.float32)
    zerof = jnp.zeros((L,), jnp.float32)

    def init_state():
        def ibody(r, _):
            for g in range(NG):
                mxb[r, pl.ds(g * L, L)] = neginf
                idb[r, pl.ds(g * L, L)] = zeroi
            return 0

        lax.fori_loop(0, 8, ibody, 0)

    def phase_a(buf, s):
        kc = s % NKC

        def rbody(r, _):
            mv = [mxb[r, pl.ds(g * L, L)] for g in range(NG)]
            iv = [idb[r, pl.ds(g * L, L)] for g in range(NG)]
            for cc in range(KC):
                cvec = jnp.full((L,), kc * KC + cc, jnp.int32)
                for g in range(NG):
                    val = buf[cc, r, pl.ds(g * L, L)]
                    gt = val > mv[g]
                    iv[g] = jnp.where(gt, cvec, iv[g])
                    mv[g] = jnp.maximum(val, mv[g])
            for g in range(NG):
                mxb[r, pl.ds(g * L, L)] = mv[g]
                idb[r, pl.ds(g * L, L)] = iv[g]
            @pl.when(kc == NKC - 1)
            def _():
                for g in range(NG):
                    mfb[r, pl.ds(g * L, L)] = jnp.where(
                        iv[g] == tgt, ones, zerof
                    )

            return 0

        lax.fori_loop(0, 8, rbody, 0)

    def phase_b(buf, s):
        item = wid * NITEM + s // (2 * NKC)
        kc = s % NKC
        b = item // RT

        def rbody(r, carry):
            accs = list(carry)
            mf = [mfb[r, pl.ds(g * L, L)] for g in range(NG)]
            for cc in range(KC):
                a = accs[cc]
                for g in range(NG):
                    a = a + buf[cc, r, pl.ds(g * L, L)] * mf[g]
                accs[cc] = a
            return tuple(accs)

        accs = lax.fori_loop(0, 8, rbody, tuple(zerof for _ in range(KC)))

        base = b * C + kc * KC
        for cc in range(KC):
            rr = base + cc
            plsc.addupdate(
                acc.at[rr // 64, (rr // 8) % 8, pl.ds((cc % 8) * L, L)],
                accs[cc],
            )

    copy_for(0, buf0, sem0).start()

    def step(s, buf, sem):
        ph = (s % (2 * NKC)) // NKC

        @pl.when(s % (2 * NKC) == 0)
        def _():
            init_state()

        @pl.when(ph == 0)
        def _():
            phase_a(buf, s)

        @pl.when(ph == 1)
        def _():
            phase_b(buf, s)

    def loop_body(j, _):
        s0 = 2 * j
        s1 = s0 + 1
        s2 = s0 + 2
        copy_for(s0, buf0, sem0).wait()
        copy_for(s1, buf1, sem1).start()
        step(s0, buf0, sem0)
        copy_for(s1, buf1, sem1).wait()

        @pl.when(s2 < NSEQ)
        def _():
            copy_for(s2, buf0, sem0).start()

        step(s1, buf1, sem1)
        return 0

    lax.fori_loop(0, NSEQ // 2, loop_body, 0)

    pltpu.sync_copy(acc, out_hbm.at[wid])


@jax.jit
def _class_filter_sc(x3, tvec16):
    mesh = plsc.VectorSubcoreMesh(core_axis_name="c", subcore_axis_name="s")
    partials = pl.kernel(
        _kernel_body,
        out_type=jax.ShapeDtypeStruct((NW, AT, 8, 128), jnp.float32),
        mesh=mesh,
        scratch_types=[
            pltpu.VMEM((KC, 8, WID), jnp.float32),
            pltpu.VMEM((KC, 8, WID), jnp.float32),
            pltpu.VMEM((AT, 8, 128), jnp.float32),
            pltpu.VMEM((8, WID), jnp.float32),
            pltpu.VMEM((8, WID), jnp.int32),
            pltpu.VMEM((8, WID), jnp.float32),
            pltpu.VMEM((L,), jnp.int32),
            pltpu.SemaphoreType.DMA,
            pltpu.SemaphoreType.DMA,
        ],
    )(x3, tvec16)
    return jnp.sum(partials.reshape(NW, B, C, L), axis=(0, 3))


def kernel(logits_batch, target):
    x3 = logits_batch.reshape(B * C, H, WID)
    tvec16 = jnp.full((L,), target, jnp.int32)
    return _class_filter_sc(x3, tvec16)
